# baseline (speedup 1.0000x reference)
import functools

import jax
import jax.numpy as jnp
from jax import lax
from jax.experimental import pallas as pl
from jax.experimental.pallas import tpu as pltpu
from jax.experimental.pallas import tpu_sc as plsc


_B, _N = 16384, 128
_NV = _N // 16
_NW = 32
_RPW = _B // _NW
_CH = 256
_NCH = _RPW // _CH
_R = 256


def _vsort(p, desc):
    k, v = p
    ks, vs = plsc.sort_key_val(k, v)
    if desc:
        ks = lax.rev(ks, (0,))
        vs = lax.rev(vs, (0,))
    return ks, vs


def _cmpx(a, b, desc):
    (ka, va), (kb, vb) = a, b
    swap = (ka > kb) if not desc else (ka < kb)
    lo = (jnp.where(swap, kb, ka), jnp.where(swap, vb, va))
    hi = (jnp.where(swap, ka, kb), jnp.where(swap, va, vb))
    return lo, hi


def _bmerge(lst, desc):
    n = len(lst)
    if n == 1:
        return [_vsort(lst[0], desc)]
    h = n // 2
    lst = list(lst)
    for i in range(h):
        lst[i], lst[i + h] = _cmpx(lst[i], lst[i + h], desc)
    return _bmerge(lst[:h], desc) + _bmerge(lst[h:], desc)


def _bsort(lst, desc):
    n = len(lst)
    if n == 1:
        return [_vsort(lst[0], desc)]
    h = n // 2
    return _bmerge(_bsort(lst[:h], not desc) + _bsort(lst[h:], desc), desc)


def _row_body(g, carry, sv, dv):
    for u in range(2):
        base = (g * 2 + u) * _N
        pairs = []
        for j in range(_NV):
            k = -sv[pl.ds(base + 16 * j, 16)]
            v = dv[pl.ds(base + 16 * j, 16)]
            pairs.append((k, v))
        srt = _bsort(pairs, False)
        for j in range(_NV):
            sv[pl.ds(base + 16 * j, 16)] = srt[j][0]
            dv[pl.ds(base + 16 * j, 16)] = srt[j][1]
    return carry


@functools.partial(
    pl.kernel,
    mesh=plsc.VectorSubcoreMesh(core_axis_name="c", subcore_axis_name="s"),
    out_type=(jax.ShapeDtypeStruct((_B * _N,), jnp.float32),
              jax.ShapeDtypeStruct((_B * _N,), jnp.float32)),
    scratch_types=[
        pltpu.VMEM((_CH * _N,), jnp.float32),
        pltpu.VMEM((_CH * _N,), jnp.float32),
    ],
    compiler_params=pltpu.CompilerParams(needs_layout_passes=False),
)
def _sc_sort(s_hbm, d_hbm, k_out, d_out, sv, dv):
    wid = lax.axis_index("s") * 2 + lax.axis_index("c")
    for ch in range(_NCH):
        rbase = wid * _RPW + ch * _CH
        pltpu.sync_copy(s_hbm.at[pl.ds(rbase * _N, _CH * _N)], sv)
        pltpu.sync_copy(d_hbm.at[pl.ds(rbase * _N, _CH * _N)], dv)
        body = functools.partial(_row_body, sv=sv, dv=dv)
        lax.fori_loop(0, _CH // 2, body, jnp.int32(0))
        pltpu.sync_copy(sv, k_out.at[pl.ds(rbase * _N, _CH * _N)])
        pltpu.sync_copy(dv, d_out.at[pl.ds(rbase * _N, _CH * _N)])


def _tail_body(k_ref, d_ref, out_ref):
    k = k_ref[...]
    dd = d_ref[...]
    lane = jax.lax.broadcasted_iota(jnp.int32, (_R, _N), 1)

    tri_r = jax.lax.broadcasted_iota(jnp.int32, (_N, _N), 0)
    tri_c = jax.lax.broadcasted_iota(jnp.int32, (_N, _N), 1)
    tri = (tri_r <= tri_c).astype(jnp.bfloat16)
    c = jax.lax.dot_general(
        dd.astype(jnp.bfloat16), tri, (((1,), (0,)), ((), ())),
        preferred_element_type=jnp.float32)

    w = lane * 256 + c.astype(jnp.int32)
    for s in (1, 2, 4, 8, 16, 32, 64):
        rw = pltpu.roll(w, _N - s, axis=1)
        rk = pltpu.roll(k, _N - s, axis=1)
        cond = (rk == k) & (lane < (_N - s))
        w = jnp.where(cond, rw, w)

    inter = (w & 255).astype(jnp.float32)
    e = (w >> 8).astype(jnp.float32)
    ell = c[:, _N - 1:_N]
    union = e + 1.0 + ell - inter
    jac = inter / union

    maxv = jnp.max(jac, axis=1, keepdims=True)
    idx = jnp.min(jnp.where(jac >= maxv, lane, _N), axis=1)
    out = jnp.concatenate(
        [(idx + 1)[:, None], ell.astype(jnp.int32)], axis=1)
    out_ref[...] = out


@jax.jit
def kernel(scores, drugset_mul_hot):
    s_flat = scores.reshape(-1)
    d_flat = drugset_mul_hot.astype(jnp.float32).reshape(-1)
    k_srt, d_srt = _sc_sort(s_flat, d_flat)
    return pl.pallas_call(
        _tail_body,
        grid=(_B // _R,),
        in_specs=[
            pl.BlockSpec((_R, _N), lambda i: (i, 0)),
            pl.BlockSpec((_R, _N), lambda i: (i, 0)),
        ],
        out_specs=pl.BlockSpec((_R, 2), lambda i: (i, 0)),
        out_shape=jax.ShapeDtypeStruct((_B, 2), jnp.int32),
    )(k_srt.reshape(_B, _N), d_srt.reshape(_B, _N))


# speedup vs baseline: 1.1140x; 1.1140x over previous
import functools

import jax
import jax.numpy as jnp
from jax import lax
from jax.experimental import pallas as pl
from jax.experimental.pallas import tpu as pltpu
from jax.experimental.pallas import tpu_sc as plsc


_B, _N = 16384, 128
_NV = _N // 16
_NW = 32
_RPW = _B // _NW
_CH = 256
_NCH = _RPW // _CH
_R = 256


def _vsort(p, desc):
    k, v = p
    ks, vs = plsc.sort_key_val(k, v)
    if desc:
        ks = lax.rev(ks, (0,))
        vs = lax.rev(vs, (0,))
    return ks, vs


def _cmpx(a, b, desc):
    (ka, va), (kb, vb) = a, b
    swap = (ka > kb) if not desc else (ka < kb)
    lo = (jnp.where(swap, kb, ka), jnp.where(swap, vb, va))
    hi = (jnp.where(swap, ka, kb), jnp.where(swap, va, vb))
    return lo, hi


def _bmerge(lst, desc):
    n = len(lst)
    if n == 1:
        return [_vsort(lst[0], desc)]
    h = n // 2
    lst = list(lst)
    for i in range(h):
        lst[i], lst[i + h] = _cmpx(lst[i], lst[i + h], desc)
    return _bmerge(lst[:h], desc) + _bmerge(lst[h:], desc)


def _bsort(lst, desc):
    n = len(lst)
    if n == 1:
        return [_vsort(lst[0], desc)]
    h = n // 2
    return _bmerge(_bsort(lst[:h], not desc) + _bsort(lst[h:], desc), desc)


def _row_body(r, carry, sv, dv):
    base = r * _N
    pairs = []
    for j in range(_NV):
        k = -sv[pl.ds(base + 16 * j, 16)]
        v = dv[pl.ds(base + 16 * j, 16)]
        pairs.append((k, v))
    srt = _bsort(pairs, False)
    for j in range(_NV):
        sv[pl.ds(base + 16 * j, 16)] = srt[j][0]
        dv[pl.ds(base + 16 * j, 16)] = srt[j][1]
    return carry


@functools.partial(
    pl.kernel,
    mesh=plsc.VectorSubcoreMesh(core_axis_name="c", subcore_axis_name="s"),
    out_type=(jax.ShapeDtypeStruct((_B * _N,), jnp.float32),
              jax.ShapeDtypeStruct((_B * _N,), jnp.float32)),
    scratch_types=[
        pltpu.VMEM((_CH * _N,), jnp.float32),
        pltpu.VMEM((_CH * _N,), jnp.float32),
    ],
    compiler_params=pltpu.CompilerParams(needs_layout_passes=False),
)
def _sc_sort(s_hbm, d_hbm, k_out, d_out, sv, dv):
    wid = lax.axis_index("s") * 2 + lax.axis_index("c")
    for ch in range(_NCH):
        rbase = wid * _RPW + ch * _CH
        pltpu.sync_copy(s_hbm.at[pl.ds(rbase * _N, _CH * _N)], sv)
        pltpu.sync_copy(d_hbm.at[pl.ds(rbase * _N, _CH * _N)], dv)
        body = functools.partial(_row_body, sv=sv, dv=dv)
        lax.fori_loop(0, _CH, body, jnp.int32(0))
        pltpu.sync_copy(sv, k_out.at[pl.ds(rbase * _N, _CH * _N)])
        pltpu.sync_copy(dv, d_out.at[pl.ds(rbase * _N, _CH * _N)])


def _tail_body(k_ref, d_ref, out_ref):
    k = k_ref[...]
    dd = d_ref[...]
    lane = jax.lax.broadcasted_iota(jnp.int32, (_R, _N), 1)

    tri_r = jax.lax.broadcasted_iota(jnp.int32, (_N, _N), 0)
    tri_c = jax.lax.broadcasted_iota(jnp.int32, (_N, _N), 1)
    tri = (tri_r <= tri_c).astype(jnp.bfloat16)
    c = jax.lax.dot_general(
        dd.astype(jnp.bfloat16), tri, (((1,), (0,)), ((), ())),
        preferred_element_type=jnp.float32)

    w = lane * 256 + c.astype(jnp.int32)
    for s in (1, 2, 4, 8, 16, 32, 64):
        rw = pltpu.roll(w, _N - s, axis=1)
        rk = pltpu.roll(k, _N - s, axis=1)
        cond = (rk == k) & (lane < (_N - s))
        w = jnp.where(cond, rw, w)

    inter = (w & 255).astype(jnp.float32)
    e = (w >> 8).astype(jnp.float32)
    ell = c[:, _N - 1:_N]
    union = e + 1.0 + ell - inter
    jac = inter / union

    maxv = jnp.max(jac, axis=1, keepdims=True)
    idx = jnp.min(jnp.where(jac >= maxv, lane, _N), axis=1)
    out = jnp.concatenate(
        [(idx + 1)[:, None], ell.astype(jnp.int32)], axis=1)
    out_ref[...] = out


@jax.jit
def kernel(scores, drugset_mul_hot):
    s_flat = scores.reshape(-1)
    d_flat = drugset_mul_hot.astype(jnp.float32).reshape(-1)
    k_srt, d_srt = _sc_sort(s_flat, d_flat)
    return pl.pallas_call(
        _tail_body,
        grid=(_B // _R,),
        in_specs=[
            pl.BlockSpec((_R, _N), lambda i: (i, 0)),
            pl.BlockSpec((_R, _N), lambda i: (i, 0)),
        ],
        out_specs=pl.BlockSpec((_R, 2), lambda i: (i, 0)),
        out_shape=jax.ShapeDtypeStruct((_B, 2), jnp.int32),
    )(k_srt.reshape(_B, _N), d_srt.reshape(_B, _N))
